# baseline (speedup 1.0000x reference)
import jax
import jax.numpy as jnp
from jax import lax
from jax.experimental import pallas as pl
from jax.experimental.pallas import tpu as pltpu
from jax.experimental.pallas import tpu_sc as plsc

F32 = jnp.float32
_BN_SCALE = 1.0 / (1.0 + 1e-5) ** 0.5

_NC = 2
_NS = 16
_C = 80
_NBUF = 3


def _make_edge_kernel(n_nodes, n_edges, width, n_pass, with_cnt):
    mesh = plsc.VectorSubcoreMesh(core_axis_name="c", subcore_axis_name="s")
    edges_per_tile = n_edges // (_NC * _NS)
    chunks = edges_per_tile // _C
    assert chunks * _C == edges_per_tile and (chunks - 2) % _NBUF == 0
    trips = (chunks - 2) // _NBUF
    n_pad = -(-n_nodes // (_NS * _C)) * (_NS * _C)
    rows_per_tile = n_pad // _NS
    zcopies = rows_per_tile // _C
    groups = width // 16
    pack = 2
    cpk = _C // pack
    cw = 128
    halfsel = width == 32

    out_type = [jax.ShapeDtypeStruct((_NC, n_pad, width), F32)
                for _ in range(n_pass)]
    if with_cnt:
        out_type.append(jax.ShapeDtypeStruct((_NC, n_pad, 16), F32))

    scratch = dict(
        dst_t=pltpu.VMEM((chunks, _C), jnp.int32),
        src_t=pltpu.VMEM((chunks, _C), jnp.int32),
        a_buf=pltpu.VMEM((_NBUF, _C, width), F32),
        b_buf=pltpu.VMEM((_NBUF, _C, width), F32),
        c_buf=pltpu.VMEM((_NBUF, cpk, cw), F32),
        s_sh=pltpu.VMEM_SHARED((n_pad, width), F32),
    )
    for nm in ('a', 'b', 'c', 'sc'):
        for sl in range(_NBUF):
            scratch[f'sem_{nm}{sl}'] = pltpu.SemaphoreType.DMA
    if with_cnt:
        scratch.update(
            ones_b=pltpu.VMEM((_C, 16), F32),
            zcnt=pltpu.VMEM((_C, 16), F32),
            cnt_sh=pltpu.VMEM_SHARED((n_pad, 16), F32),
            sem_cnt=pltpu.SemaphoreType.DMA,
        )

    def body(*refs, dst_t, src_t, a_buf, b_buf, c_buf, s_sh,
             ones_b=None, zcnt=None, cnt_sh=None, sem_cnt=None, **sems):
        abc = refs[:3 * n_pass]
        dst2, src2 = refs[3 * n_pass:3 * n_pass + 2]
        s_outs = refs[3 * n_pass + 2:3 * n_pass + 2 + n_pass]
        cnt_out = refs[-1] if with_cnt else None

        cid = lax.axis_index("c")
        sid = lax.axis_index("s")
        tc0 = cid * (n_edges // _NC // _C) + sid * chunks
        tbase = tc0 * _C
        row0 = sid * rows_per_tile

        pltpu.sync_copy(dst2.at[pl.ds(tc0, chunks)], dst_t)
        pltpu.sync_copy(src2.at[pl.ds(tc0, chunks)], src_t)

        for p in range(n_pass):
            a_hbm, b_hbm, cb_hbm = abc[3 * p:3 * p + 3]
            s_out = s_outs[p]
            cnt_pass = with_cnt and p == 0

            def zrow(i, _):
                for j in range(groups):
                    a_buf[_NBUF - 1, i, pl.ds(j * 16, 16)] = jnp.zeros((16,), F32)
                if cnt_pass:
                    zcnt[i, :] = jnp.zeros((16,), F32)
                    ones_b[i, :] = jnp.ones((16,), F32)
                return 0

            lax.fori_loop(0, _C, zrow, 0)
            for k in range(zcopies):
                pltpu.sync_copy(a_buf.at[_NBUF - 1],
                                s_sh.at[pl.ds(row0 + k * _C, _C)])
                if cnt_pass:
                    pltpu.sync_copy(zcnt, cnt_sh.at[pl.ds(row0 + k * _C, _C)])
            plsc.subcore_barrier()

            def issue(kk, t):
                pltpu.async_copy(a_hbm.at[dst_t.at[kk]], a_buf.at[t],
                                 sems[f'sem_a{t}'])
                pltpu.async_copy(b_hbm.at[src_t.at[kk]], b_buf.at[t],
                                 sems[f'sem_b{t}'])
                if halfsel:
                    crow = cpk * ((tbase // _C + kk) // 2)
                else:
                    crow = tbase // 2 + kk * cpk
                pltpu.async_copy(cb_hbm.at[pl.ds(crow, cpk)],
                                 c_buf.at[t], sems[f'sem_c{t}'])

            def wait_data(s):
                dummy = a_hbm.at[pl.ds(0, _C)]
                pltpu.make_async_copy(dummy, a_buf.at[s], sems[f'sem_a{s}']).wait()
                pltpu.make_async_copy(dummy, b_buf.at[s], sems[f'sem_b{s}']).wait()
                pltpu.make_async_copy(cb_hbm.at[pl.ds(0, cpk)], c_buf.at[s],
                                      sems[f'sem_c{s}']).wait()

            def wait_scatter(t):
                pltpu.make_async_copy(a_hbm.at[pl.ds(0, _C)], a_buf.at[t],
                                      sems[f'sem_sc{t}']).wait()

            def wait_cnt():
                pltpu.make_async_copy(cnt_out.at[0, pl.ds(0, _C)], zcnt,
                                      sem_cnt).wait()

            def process(k, s, do_issue):
                wait_data(s)
                cb0 = (tbase // _C + k) % 2 * 2 if halfsel else 0

                def rowf(ip, _):
                    for u in range(pack):
                        i = u * cpk + ip
                        for j in range(groups):
                            sl = pl.ds(j * 16, 16)
                            cl = pl.ds((cb0 + u) * width + j * 16, 16)
                            v = (a_buf[s, i, sl] + b_buf[s, i, sl]
                                 + c_buf[s, ip, cl])
                            a_buf[s, i, sl] = jnp.maximum(v, 0.0)
                    return 0

                lax.fori_loop(0, cpk, rowf, 0)
                pltpu.async_copy(a_buf.at[s], s_sh.at[dst_t.at[k]],
                                 sems[f'sem_sc{s}'], add=True)
                if cnt_pass:
                    wait_cnt()
                    pltpu.async_copy(ones_b, cnt_sh.at[dst_t.at[k]], sem_cnt,
                                     add=True)
                if do_issue:
                    t = (s + 2) % _NBUF
                    wait_scatter(t)
                    issue(k + 2, t)

            issue(0, 0)
            issue(1, 1)
            pltpu.async_copy(a_buf.at[_NBUF - 1], s_sh.at[dst_t.at[0]],
                             sems[f'sem_sc{_NBUF - 1}'], add=True)
            if cnt_pass:
                pltpu.async_copy(zcnt, cnt_sh.at[dst_t.at[0]], sem_cnt,
                                 add=True)

            def trip(g, _):
                for j in range(_NBUF):
                    process(_NBUF * g + j, j, True)
                return 0

            lax.fori_loop(0, trips, trip, 0)
            process(chunks - 2, (chunks - 2) % _NBUF, False)
            process(chunks - 1, (chunks - 1) % _NBUF, False)
            for t in range(_NBUF):
                wait_scatter(t)
            if cnt_pass:
                wait_cnt()
            plsc.subcore_barrier()

            pltpu.sync_copy(s_sh.at[pl.ds(row0, rows_per_tile)],
                            s_out.at[cid, pl.ds(row0, rows_per_tile)])
            if cnt_pass:
                pltpu.sync_copy(cnt_sh.at[pl.ds(row0, rows_per_tile)],
                                cnt_out.at[cid, pl.ds(row0, rows_per_tile)])

    return pl.kernel(body, out_type=tuple(out_type), mesh=mesh,
                     scratch_types=scratch,
                     compiler_params=pltpu.CompilerParams(
                         use_tc_tiling_on_sc=False))


def _prep_call(x, ne_W, ne_b, w1s, nb):
    n, df = x.shape
    bn = n // nb

    def body(*refs):
        x_ref, w_ref, b_ref = refs[:3]
        ws = refs[3:3 + len(w1s)]
        outs = refs[3 + len(w1s):]
        h = jnp.maximum(x_ref[...] @ w_ref[...] + b_ref[...], 0.0)
        for w, o in zip(ws, outs):
            o[...] = h @ w[...]

    full = lambda a: pl.BlockSpec(a.shape, lambda i: (0,) * a.ndim)
    return pl.pallas_call(
        body,
        grid=(nb,),
        in_specs=[pl.BlockSpec((bn, df), lambda i: (i, 0)),
                  full(ne_W), full(ne_b)] + [full(w) for w in w1s],
        out_specs=[pl.BlockSpec((bn, w.shape[1]), lambda i: (i, 0))
                   for w in w1s],
        out_shape=[jax.ShapeDtypeStruct((n, w.shape[1]), F32) for w in w1s],
    )(x, ne_W, ne_b, *w1s)


def _edgeprep_call(edge_attr, ee_W, ee_b, weas, b1s, packs, nb):
    e, de = edge_attr.shape

    def body(*refs):
        ea_ref, w_ref, b_ref = refs[:3]
        k = len(weas)
        ws = refs[3:3 + k]
        bs = refs[3 + k:3 + 2 * k]
        outs = refs[3 + 2 * k:]
        ea = jnp.maximum(ea_ref[...] @ w_ref[...] + b_ref[...], 0.0)
        for w, b, o, pk in zip(ws, bs, outs, packs):
            cb = ea @ w[...] + b[...]
            grp = 40 * pk
            ng = cb.shape[0] // grp
            stacks = [
                jnp.concatenate(
                    [cb[grp * m + 40 * q:grp * m + 40 * q + 40]
                     for m in range(ng)], 0)
                for q in range(pk)]
            o[...] = jnp.concatenate(stacks, axis=1)

    be = e // nb
    full = lambda a: pl.BlockSpec(a.shape, lambda i: (0,) * a.ndim)
    return pl.pallas_call(
        body,
        grid=(nb,),
        in_specs=[pl.BlockSpec((be, de), lambda i: (i, 0)),
                  full(ee_W), full(ee_b)]
                 + [full(w) for w in weas] + [full(b) for b in b1s],
        out_specs=[pl.BlockSpec((be // pk, 128), lambda i: (i, 0))
                   for pk in packs],
        out_shape=[jax.ShapeDtypeStruct((e // pk, 128), F32) for pk in packs],
    )(edge_attr, ee_W, ee_b, *weas, *b1s)


def _post_call(s_list, cnt_p, w2_list, b2, g, bb, wa, wb, n, nb):
    ns = len(s_list)
    bn_ = n // nb
    h_next = wa.shape[1]

    def body(*refs):
        s_refs = refs[:ns]
        c_ref = refs[ns]
        w2_refs = refs[ns + 1:2 * ns + 1]
        b2_ref, g_ref, bb_ref, wa_ref, wb_ref = refs[2 * ns + 1:2 * ns + 6]
        a_ref, b_ref = refs[2 * ns + 6:]
        v = s_refs[0][0] + s_refs[0][1]
        acc = v @ w2_refs[0][...]
        for sr, wr in zip(s_refs[1:], w2_refs[1:]):
            acc += (sr[0] + sr[1]) @ wr[...]
        cnt = (c_ref[0] + c_ref[1])[:, 0:1]
        deg = jnp.maximum(cnt, 1.0)
        has = (cnt > 0.0).astype(F32)
        v = acc / deg + b2_ref[...] * has
        v = g_ref[...] * (v * _BN_SCALE) + bb_ref[...]
        hn = jnp.maximum(v, 0.0)
        a_ref[...] = hn @ wa_ref[...]
        b_ref[...] = hn @ wb_ref[...]

    full = lambda a: pl.BlockSpec(a.shape, lambda i: (0,) * a.ndim)
    return pl.pallas_call(
        body,
        grid=(nb,),
        in_specs=[pl.BlockSpec((2, bn_, s.shape[2]), lambda i: (0, i, 0))
                  for s in s_list]
                 + [pl.BlockSpec((2, bn_, 16), lambda i: (0, i, 0))]
                 + [full(w) for w in w2_list]
                 + [full(b2), full(g), full(bb), full(wa), full(wb)],
        out_specs=[pl.BlockSpec((bn_, h_next), lambda i: (i, 0))] * 2,
        out_shape=[jax.ShapeDtypeStruct((n, h_next), F32)] * 2,
    )(*s_list, cnt_p, *w2_list, b2, g, bb, wa, wb)


def _final_call(s_p, cnt_p, batch3, w2, b2, g, bb, fc_W, fc_b, n_graphs, n, nb):
    h = s_p.shape[2]
    bn_ = n // nb
    dout = fc_W.shape[1]
    h3w = w2.shape[1]

    def body(s_ref, c_ref, b3_ref, w2_ref, b2_ref, g_ref, bb_ref, fw_ref,
             fb_ref, o_ref, gsum, gcnt):
        i = pl.program_id(0)

        @pl.when(i == 0)
        def _():
            gsum[...] = jnp.zeros_like(gsum)
            gcnt[...] = jnp.zeros_like(gcnt)

        s = s_ref[0] + s_ref[1]
        cnt = (c_ref[0] + c_ref[1])[:, 0:1]
        deg = jnp.maximum(cnt, 1.0)
        has = (cnt > 0.0).astype(F32)
        v = (s @ w2_ref[...]) / deg + b2_ref[...] * has
        v = g_ref[...] * (v * _BN_SCALE) + bb_ref[...]
        h3 = jnp.maximum(v, 0.0)

        seg = jnp.broadcast_to(b3_ref[0], (n_graphs, bn_))
        oh = (seg == lax.broadcasted_iota(jnp.int32, (n_graphs, bn_), 0)
              ).astype(F32)
        gsum[...] += oh @ h3
        gcnt[...] += jnp.broadcast_to(jnp.sum(oh, axis=1, keepdims=True),
                                      gcnt.shape)

        @pl.when(i == nb - 1)
        def _():
            pooled = gsum[...] / jnp.maximum(gcnt[...], 1.0)
            o_ref[...] = jax.nn.sigmoid(pooled @ fw_ref[...] + fb_ref[...])

    full = lambda a: pl.BlockSpec(a.shape, lambda i: (0,) * a.ndim)
    return pl.pallas_call(
        body,
        grid=(nb,),
        in_specs=[pl.BlockSpec((2, bn_, h), lambda i: (0, i, 0)),
                  pl.BlockSpec((2, bn_, 16), lambda i: (0, i, 0)),
                  pl.BlockSpec((1, 1, bn_), lambda i: (i, 0, 0)),
                  full(w2), full(b2), full(g), full(bb), full(fc_W),
                  full(fc_b)],
        out_specs=pl.BlockSpec((n_graphs, dout), lambda i: (0, 0)),
        out_shape=jax.ShapeDtypeStruct((n_graphs, dout), F32),
        scratch_shapes=[pltpu.VMEM((n_graphs, h3w), F32),
                        pltpu.VMEM((n_graphs, 16), F32)],
    )(s_p, cnt_p, batch3, w2, b2, g, bb, fc_W, fc_b)


def kernel(x, edge_index, edge_attr, batch, params):
    p = params
    n, df = x.shape
    e = edge_index.shape[1]
    n_graphs = 256
    src = edge_index[0]
    dst = edge_index[1]
    dst2 = dst.reshape(e // _C, _C)
    src2 = src.reshape(e // _C, _C)
    r1 = lambda a: a.reshape(1, -1)

    dh1 = p['ne_W'].shape[1]
    h1 = p['c1_W1'].shape[1]
    hh = h1 // 2
    dh2 = p['c1_W2'].shape[1]
    h2 = p['c2_W1'].shape[1]
    dh3 = p['c2_W2'].shape[1]
    h3 = p['c3_W1'].shape[1]

    w1d, w1s, w1e = (p['c1_W1'][:dh1], p['c1_W1'][dh1:2 * dh1],
                     p['c1_W1'][2 * dh1:])
    a1a, a1b, b1a, b1b = _prep_call(
        x, p['ne_W'], r1(p['ne_b']),
        [w1d[:, :hh], w1d[:, hh:], w1s[:, :hh], w1s[:, hh:]], nb=10)
    cb1a, cb1b = _edgeprep_call(
        edge_attr, p['ee_W'], r1(p['ee_b']),
        [w1e[:, :hh], w1e[:, hh:]],
        [r1(p['c1_b1'][:hh]), r1(p['c1_b1'][hh:])], packs=[2, 2], nb=100)
    cb2, cb3 = _edgeprep_call(
        edge_attr, p['ee_W'], r1(p['ee_b']),
        [p['c2_W1'][2 * dh2:], p['c3_W1'][2 * dh3:]],
        [r1(p['c2_b1']), r1(p['c3_b1'])], packs=[2, 4], nb=100)

    s1a, s1b, cnt = _make_edge_kernel(n, e, hh, 2, True)(
        a1a, b1a, cb1a, a1b, b1b, cb1b, dst2, src2)
    a2, b2 = _post_call([s1a, s1b], cnt,
                        [p['c1_W2'][:hh], p['c1_W2'][hh:]],
                        r1(p['c1_b2']), r1(p['bn1_g']), r1(p['bn1_b']),
                        p['c2_W1'][:dh2], p['c2_W1'][dh2:2 * dh2], n=n, nb=10)

    (s2,) = _make_edge_kernel(n, e, h2, 1, False)(a2, b2, cb2, dst2, src2)
    a3, b3 = _post_call([s2], cnt, [p['c2_W2']],
                        r1(p['c2_b2']), r1(p['bn2_g']), r1(p['bn2_b']),
                        p['c3_W1'][:dh3], p['c3_W1'][dh3:2 * dh3], n=n, nb=10)

    (s3,) = _make_edge_kernel(n, e, h3, 1, False)(a3, b3, cb3, dst2, src2)
    batch3 = batch.reshape(10, 1, n // 10)
    return _final_call(s3, cnt, batch3, p['c3_W2'], r1(p['c3_b2']),
                       r1(p['bn3_g']), r1(p['bn3_b']),
                       p['fc_W'], r1(p['fc_b']), n_graphs, n=n, nb=10)


# speedup vs baseline: 1.1002x; 1.1002x over previous
import jax
import jax.numpy as jnp
from jax import lax
from jax.experimental import pallas as pl
from jax.experimental.pallas import tpu as pltpu
from jax.experimental.pallas import tpu_sc as plsc

F32 = jnp.float32
_BN_SCALE = 1.0 / (1.0 + 1e-5) ** 0.5

_NC = 2
_NS = 16
_C = 80
_NBUF = 3


def _make_edge_kernel(n_nodes, n_edges, width, n_pass, with_cnt):
    mesh = plsc.VectorSubcoreMesh(core_axis_name="c", subcore_axis_name="s")
    edges_per_tile = n_edges // (_NC * _NS)
    chunks = edges_per_tile // _C
    assert chunks * _C == edges_per_tile and (chunks - 2) % _NBUF == 0
    trips = (chunks - 2) // _NBUF
    n_pad = -(-n_nodes // (_NS * _C)) * (_NS * _C)
    rows_per_tile = n_pad // _NS
    zcopies = rows_per_tile // _C
    groups = width // 16
    pack = 2
    cpk = _C // pack
    cw = 128
    halfsel = width == 32

    out_type = [jax.ShapeDtypeStruct((_NC, n_pad, width), F32)
                for _ in range(n_pass)]
    if with_cnt:
        out_type.append(jax.ShapeDtypeStruct((_NC, n_pad, 16), F32))

    scratch = dict(
        dst_t=pltpu.VMEM((chunks, _C), jnp.int32),
        src_t=pltpu.VMEM((chunks, _C), jnp.int32),
        a_buf=pltpu.VMEM((_NBUF, _C, width), F32),
        b_buf=pltpu.VMEM((_NBUF, _C, width), F32),
        c_buf=pltpu.VMEM((_NBUF, cpk, cw), F32),
        s_sh=pltpu.VMEM_SHARED((n_pad, width), F32),
    )
    for nm in ('a', 'b', 'c', 'sc'):
        for sl in range(_NBUF):
            scratch[f'sem_{nm}{sl}'] = pltpu.SemaphoreType.DMA
    if with_cnt:
        scratch.update(
            ones_b=pltpu.VMEM((_C, 16), F32),
            zcnt=pltpu.VMEM((_C, 16), F32),
            cnt_sh=pltpu.VMEM_SHARED((n_pad, 16), F32),
            sem_cnt=pltpu.SemaphoreType.DMA,
        )

    def body(*refs, dst_t, src_t, a_buf, b_buf, c_buf, s_sh,
             ones_b=None, zcnt=None, cnt_sh=None, sem_cnt=None, **sems):
        abc = refs[:3 * n_pass]
        dst2, src2 = refs[3 * n_pass:3 * n_pass + 2]
        s_outs = refs[3 * n_pass + 2:3 * n_pass + 2 + n_pass]
        cnt_out = refs[-1] if with_cnt else None

        cid = lax.axis_index("c")
        sid = lax.axis_index("s")
        tc0 = cid * (n_edges // _NC // _C) + sid * chunks
        tbase = tc0 * _C
        row0 = sid * rows_per_tile

        pltpu.sync_copy(dst2.at[pl.ds(tc0, chunks)], dst_t)
        pltpu.sync_copy(src2.at[pl.ds(tc0, chunks)], src_t)

        for p in range(n_pass):
            a_hbm, b_hbm, cb_hbm = abc[3 * p:3 * p + 3]
            s_out = s_outs[p]
            cnt_pass = with_cnt and p == 0

            def zrow(i, _):
                for j in range(groups):
                    a_buf[_NBUF - 1, i, pl.ds(j * 16, 16)] = jnp.zeros((16,), F32)
                if cnt_pass:
                    zcnt[i, :] = jnp.zeros((16,), F32)
                    ones_b[i, :] = jnp.ones((16,), F32)
                return 0

            lax.fori_loop(0, _C, zrow, 0)
            for k in range(zcopies):
                pltpu.sync_copy(a_buf.at[_NBUF - 1],
                                s_sh.at[pl.ds(row0 + k * _C, _C)])
                if cnt_pass:
                    pltpu.sync_copy(zcnt, cnt_sh.at[pl.ds(row0 + k * _C, _C)])
            plsc.subcore_barrier()

            def issue(kk, t):
                pltpu.async_copy(a_hbm.at[dst_t.at[kk]], a_buf.at[t],
                                 sems[f'sem_a{t}'])
                pltpu.async_copy(b_hbm.at[src_t.at[kk]], b_buf.at[t],
                                 sems[f'sem_b{t}'])
                if halfsel:
                    crow = cpk * ((tbase // _C + kk) // 2)
                else:
                    crow = tbase // 2 + kk * cpk
                pltpu.async_copy(cb_hbm.at[pl.ds(crow, cpk)],
                                 c_buf.at[t], sems[f'sem_c{t}'])

            def wait_data(s):
                dummy = a_hbm.at[pl.ds(0, _C)]
                pltpu.make_async_copy(dummy, a_buf.at[s], sems[f'sem_a{s}']).wait()
                pltpu.make_async_copy(dummy, b_buf.at[s], sems[f'sem_b{s}']).wait()
                pltpu.make_async_copy(cb_hbm.at[pl.ds(0, cpk)], c_buf.at[s],
                                      sems[f'sem_c{s}']).wait()

            def wait_scatter(t):
                pltpu.make_async_copy(a_hbm.at[pl.ds(0, _C)], a_buf.at[t],
                                      sems[f'sem_sc{t}']).wait()

            def wait_cnt():
                pltpu.make_async_copy(cnt_out.at[0, pl.ds(0, _C)], zcnt,
                                      sem_cnt).wait()

            def process(k, s, do_issue):
                wait_data(s)
                cb0 = (tbase // _C + k) % 2 * 2 if halfsel else 0

                def rowf(ip, _):
                    for u in range(pack):
                        i = u * cpk + ip
                        for j in range(groups):
                            sl = pl.ds(j * 16, 16)
                            cl = pl.ds((cb0 + u) * width + j * 16, 16)
                            v = (a_buf[s, i, sl] + b_buf[s, i, sl]
                                 + c_buf[s, ip, cl])
                            a_buf[s, i, sl] = jnp.maximum(v, 0.0)
                    return 0

                lax.fori_loop(0, cpk, rowf, 0)
                pltpu.async_copy(a_buf.at[s], s_sh.at[dst_t.at[k]],
                                 sems[f'sem_sc{s}'], add=True)
                if cnt_pass:
                    wait_cnt()
                    pltpu.async_copy(ones_b, cnt_sh.at[dst_t.at[k]], sem_cnt,
                                     add=True)
                if do_issue:
                    t = (s + 2) % _NBUF
                    wait_scatter(t)
                    issue(k + 2, t)

            issue(0, 0)
            issue(1, 1)
            pltpu.async_copy(a_buf.at[_NBUF - 1], s_sh.at[dst_t.at[0]],
                             sems[f'sem_sc{_NBUF - 1}'], add=True)
            if cnt_pass:
                pltpu.async_copy(zcnt, cnt_sh.at[dst_t.at[0]], sem_cnt,
                                 add=True)

            def trip(g, _):
                for j in range(_NBUF):
                    process(_NBUF * g + j, j, True)
                return 0

            lax.fori_loop(0, trips, trip, 0)
            process(chunks - 2, (chunks - 2) % _NBUF, False)
            process(chunks - 1, (chunks - 1) % _NBUF, False)
            for t in range(_NBUF):
                wait_scatter(t)
            if cnt_pass:
                wait_cnt()
            plsc.subcore_barrier()

            pltpu.sync_copy(s_sh.at[pl.ds(row0, rows_per_tile)],
                            s_out.at[cid, pl.ds(row0, rows_per_tile)])
            if cnt_pass:
                pltpu.sync_copy(cnt_sh.at[pl.ds(row0, rows_per_tile)],
                                cnt_out.at[cid, pl.ds(row0, rows_per_tile)])

    return pl.kernel(body, out_type=tuple(out_type), mesh=mesh,
                     scratch_types=scratch,
                     compiler_params=pltpu.CompilerParams(
                         use_tc_tiling_on_sc=False))


def _prep_call(x, ne_W, ne_b, w1s, nb):
    n, df = x.shape
    bn = n // nb

    def body(*refs):
        x_ref, w_ref, b_ref = refs[:3]
        ws = refs[3:3 + len(w1s)]
        outs = refs[3 + len(w1s):]
        h = jnp.maximum(x_ref[...] @ w_ref[...] + b_ref[...], 0.0)
        for w, o in zip(ws, outs):
            o[...] = h @ w[...]

    full = lambda a: pl.BlockSpec(a.shape, lambda i: (0,) * a.ndim)
    return pl.pallas_call(
        body,
        grid=(nb,),
        in_specs=[pl.BlockSpec((bn, df), lambda i: (i, 0)),
                  full(ne_W), full(ne_b)] + [full(w) for w in w1s],
        out_specs=[pl.BlockSpec((bn, w.shape[1]), lambda i: (i, 0))
                   for w in w1s],
        out_shape=[jax.ShapeDtypeStruct((n, w.shape[1]), F32) for w in w1s],
    )(x, ne_W, ne_b, *w1s)


def _edgeprep_call(edge_attr_t, ee_W, ee_b, weas, b1s, packs, nb):
    de, e = edge_attr_t.shape

    def body(*refs):
        ea_ref, w_ref, b_ref = refs[:3]
        k = len(weas)
        ws = refs[3:3 + k]
        bs = refs[3 + k:3 + 2 * k]
        outs = refs[3 + 2 * k:]
        prod = lax.dot_general(ea_ref[...], w_ref[...],
                               (((0,), (0,)), ((), ())),
                               preferred_element_type=F32)
        ea = jnp.maximum(prod + b_ref[...], 0.0)
        for w, b, o, pk in zip(ws, bs, outs, packs):
            cb = ea @ w[...] + b[...]
            grp = 40 * pk
            ng = cb.shape[0] // grp
            stacks = [
                jnp.concatenate(
                    [cb[grp * m + 40 * q:grp * m + 40 * q + 40]
                     for m in range(ng)], 0)
                for q in range(pk)]
            o[...] = jnp.concatenate(stacks, axis=1)

    be = e // nb
    full = lambda a: pl.BlockSpec(a.shape, lambda i: (0,) * a.ndim)
    return pl.pallas_call(
        body,
        grid=(nb,),
        in_specs=[pl.BlockSpec((de, be), lambda i: (0, i)),
                  full(ee_W), full(ee_b)]
                 + [full(w) for w in weas] + [full(b) for b in b1s],
        out_specs=[pl.BlockSpec((be // pk, 128), lambda i: (i, 0))
                   for pk in packs],
        out_shape=[jax.ShapeDtypeStruct((e // pk, 128), F32) for pk in packs],
    )(edge_attr_t, ee_W, ee_b, *weas, *b1s)


def _post_call(s_list, cnt_p, w2_list, b2, g, bb, wa, wb, n, nb):
    ns = len(s_list)
    bn_ = n // nb
    h_next = wa.shape[1]

    def body(*refs):
        s_refs = refs[:ns]
        c_ref = refs[ns]
        w2_refs = refs[ns + 1:2 * ns + 1]
        b2_ref, g_ref, bb_ref, wa_ref, wb_ref = refs[2 * ns + 1:2 * ns + 6]
        a_ref, b_ref = refs[2 * ns + 6:]
        v = s_refs[0][0] + s_refs[0][1]
        acc = v @ w2_refs[0][...]
        for sr, wr in zip(s_refs[1:], w2_refs[1:]):
            acc += (sr[0] + sr[1]) @ wr[...]
        cnt = (c_ref[0] + c_ref[1])[:, 0:1]
        deg = jnp.maximum(cnt, 1.0)
        has = (cnt > 0.0).astype(F32)
        v = acc / deg + b2_ref[...] * has
        v = g_ref[...] * (v * _BN_SCALE) + bb_ref[...]
        hn = jnp.maximum(v, 0.0)
        a_ref[...] = hn @ wa_ref[...]
        b_ref[...] = hn @ wb_ref[...]

    full = lambda a: pl.BlockSpec(a.shape, lambda i: (0,) * a.ndim)
    return pl.pallas_call(
        body,
        grid=(nb,),
        in_specs=[pl.BlockSpec((2, bn_, s.shape[2]), lambda i: (0, i, 0))
                  for s in s_list]
                 + [pl.BlockSpec((2, bn_, 16), lambda i: (0, i, 0))]
                 + [full(w) for w in w2_list]
                 + [full(b2), full(g), full(bb), full(wa), full(wb)],
        out_specs=[pl.BlockSpec((bn_, h_next), lambda i: (i, 0))] * 2,
        out_shape=[jax.ShapeDtypeStruct((n, h_next), F32)] * 2,
    )(*s_list, cnt_p, *w2_list, b2, g, bb, wa, wb)


def _final_call(s_p, cnt_p, batch3, w2, b2, g, bb, fc_W, fc_b, n_graphs, n, nb):
    h = s_p.shape[2]
    bn_ = n // nb
    dout = fc_W.shape[1]
    h3w = w2.shape[1]

    def body(s_ref, c_ref, b3_ref, w2_ref, b2_ref, g_ref, bb_ref, fw_ref,
             fb_ref, o_ref, gsum, gcnt):
        i = pl.program_id(0)

        @pl.when(i == 0)
        def _():
            gsum[...] = jnp.zeros_like(gsum)
            gcnt[...] = jnp.zeros_like(gcnt)

        s = s_ref[0] + s_ref[1]
        cnt = (c_ref[0] + c_ref[1])[:, 0:1]
        deg = jnp.maximum(cnt, 1.0)
        has = (cnt > 0.0).astype(F32)
        v = (s @ w2_ref[...]) / deg + b2_ref[...] * has
        v = g_ref[...] * (v * _BN_SCALE) + bb_ref[...]
        h3 = jnp.maximum(v, 0.0)

        seg = jnp.broadcast_to(b3_ref[0], (n_graphs, bn_))
        oh = (seg == lax.broadcasted_iota(jnp.int32, (n_graphs, bn_), 0)
              ).astype(F32)
        gsum[...] += oh @ h3
        gcnt[...] += jnp.broadcast_to(jnp.sum(oh, axis=1, keepdims=True),
                                      gcnt.shape)

        @pl.when(i == nb - 1)
        def _():
            pooled = gsum[...] / jnp.maximum(gcnt[...], 1.0)
            o_ref[...] = jax.nn.sigmoid(pooled @ fw_ref[...] + fb_ref[...])

    full = lambda a: pl.BlockSpec(a.shape, lambda i: (0,) * a.ndim)
    return pl.pallas_call(
        body,
        grid=(nb,),
        in_specs=[pl.BlockSpec((2, bn_, h), lambda i: (0, i, 0)),
                  pl.BlockSpec((2, bn_, 16), lambda i: (0, i, 0)),
                  pl.BlockSpec((1, 1, bn_), lambda i: (i, 0, 0)),
                  full(w2), full(b2), full(g), full(bb), full(fc_W),
                  full(fc_b)],
        out_specs=pl.BlockSpec((n_graphs, dout), lambda i: (0, 0)),
        out_shape=jax.ShapeDtypeStruct((n_graphs, dout), F32),
        scratch_shapes=[pltpu.VMEM((n_graphs, h3w), F32),
                        pltpu.VMEM((n_graphs, 16), F32)],
    )(s_p, cnt_p, batch3, w2, b2, g, bb, fc_W, fc_b)


def kernel(x, edge_index, edge_attr, batch, params):
    p = params
    n, df = x.shape
    e = edge_index.shape[1]
    n_graphs = 256
    src = edge_index[0]
    dst = edge_index[1]
    dst2 = dst.reshape(e // _C, _C)
    src2 = src.reshape(e // _C, _C)
    r1 = lambda a: a.reshape(1, -1)

    dh1 = p['ne_W'].shape[1]
    h1 = p['c1_W1'].shape[1]
    hh = h1 // 2
    dh2 = p['c1_W2'].shape[1]
    h2 = p['c2_W1'].shape[1]
    dh3 = p['c2_W2'].shape[1]
    h3 = p['c3_W1'].shape[1]

    w1d, w1s, w1e = (p['c1_W1'][:dh1], p['c1_W1'][dh1:2 * dh1],
                     p['c1_W1'][2 * dh1:])
    a1a, a1b, b1a, b1b = _prep_call(
        x, p['ne_W'], r1(p['ne_b']),
        [w1d[:, :hh], w1d[:, hh:], w1s[:, :hh], w1s[:, hh:]], nb=10)
    edge_attr_t = edge_attr.T
    cb1a, cb1b = _edgeprep_call(
        edge_attr_t, p['ee_W'], r1(p['ee_b']),
        [w1e[:, :hh], w1e[:, hh:]],
        [r1(p['c1_b1'][:hh]), r1(p['c1_b1'][hh:])], packs=[2, 2], nb=100)
    cb2, cb3 = _edgeprep_call(
        edge_attr_t, p['ee_W'], r1(p['ee_b']),
        [p['c2_W1'][2 * dh2:], p['c3_W1'][2 * dh3:]],
        [r1(p['c2_b1']), r1(p['c3_b1'])], packs=[2, 4], nb=100)

    s1a, s1b, cnt = _make_edge_kernel(n, e, hh, 2, True)(
        a1a, b1a, cb1a, a1b, b1b, cb1b, dst2, src2)
    a2, b2 = _post_call([s1a, s1b], cnt,
                        [p['c1_W2'][:hh], p['c1_W2'][hh:]],
                        r1(p['c1_b2']), r1(p['bn1_g']), r1(p['bn1_b']),
                        p['c2_W1'][:dh2], p['c2_W1'][dh2:2 * dh2], n=n, nb=10)

    (s2,) = _make_edge_kernel(n, e, h2, 1, False)(a2, b2, cb2, dst2, src2)
    a3, b3 = _post_call([s2], cnt, [p['c2_W2']],
                        r1(p['c2_b2']), r1(p['bn2_g']), r1(p['bn2_b']),
                        p['c3_W1'][:dh3], p['c3_W1'][dh3:2 * dh3], n=n, nb=10)

    (s3,) = _make_edge_kernel(n, e, h3, 1, False)(a3, b3, cb3, dst2, src2)
    batch3 = batch.reshape(10, 1, n // 10)
    return _final_call(s3, cnt, batch3, p['c3_W2'], r1(p['c3_b2']),
                       r1(p['bn3_g']), r1(p['bn3_b']),
                       p['fc_W'], r1(p['fc_b']), n_graphs, n=n, nb=10)
